# baseline (speedup 1.0000x reference)
import functools

import jax
import jax.numpy as jnp
from jax import lax
from jax.experimental import pallas as pl
from jax.experimental.pallas import tpu as pltpu
from jax.experimental.pallas import tpu_sc as plsc

B = 128
L = 32768
D = 2048
ND = 4096

_NC = 2
_NS = 16
_NW = _NC * _NS
_SH = L // _NS
_ROWS = B // _NW
_NCH = ND // 16
_JB = ND // _NS


def _route_body(idx_hbm, x_hbm, keep_hbm, win_hbm, xg_hbm,
                idx_v, shard_v, winpart_v, keep_v, tmp_v, acc_v,
                xrow0_v, xrow1_v, xrow2_v, xgrow_v,
                sem0, sem1, sem2, shared_win):
    c = lax.axis_index("c")
    s = lax.axis_index("s")
    base = s * _SH
    wid = s * _NC + c
    b0 = wid * _ROWS

    bufs = (xrow0_v, xrow1_v, xrow2_v)
    sems = (sem0, sem1, sem2)
    copies = [pltpu.make_async_copy(x_hbm.at[b0 + r], bufs[r % 3], sems[r % 3])
              for r in range(_ROWS)]
    for r in range(min(3, _ROWS)):
        copies[r].start()

    pltpu.sync_copy(idx_hbm, idx_v)

    @plsc.parallel_loop(0, _SH // 16, unroll=8)
    def _init_loop(i):
        shard_v[pl.ds(i * 16, 16)] = jnp.full((16,), -1, jnp.int32)

    def chunk_body(ci, carry):
        k16 = idx_v[pl.ds(ci * 16, 16)]
        j16 = lax.iota(jnp.int32, 16) + ci * 16
        m = (k16 >= base) & (k16 < base + _SH)
        loc = jnp.clip(k16 - base, 0, _SH - 1)
        cand = jnp.where(m, j16, -1)
        for _ in range(3):
            g = plsc.load_gather(shard_v, [loc], mask=m)
            need = m & (cand > g)
            plsc.store_scatter(shard_v, [loc], cand, mask=need)
        return carry
    lax.fori_loop(0, _NCH, chunk_body, 0, unroll=4)

    koff = c * (_SH // 2)

    @plsc.parallel_loop(0, _SH // 32, unroll=8)
    def _keep_loop(i):
        a16 = shard_v[pl.ds(koff + i * 16, 16)]
        keep_v[pl.ds(i * 16, 16)] = jnp.where(
            a16 == -1, jnp.float32(1.0), jnp.float32(0.0))

    pltpu.sync_copy(keep_v, keep_hbm.at[pl.ds(base + koff, _SH // 2)])

    @plsc.parallel_loop(0, _NCH, unroll=8)
    def _winpart_loop(ci):
        k16 = idx_v[pl.ds(ci * 16, 16)]
        j16 = lax.iota(jnp.int32, 16) + ci * 16
        m = (k16 >= base) & (k16 < base + _SH)
        loc = jnp.clip(k16 - base, 0, _SH - 1)
        g = plsc.load_gather(shard_v, [loc], mask=m)
        w = m & (g == j16)
        winpart_v[pl.ds(ci * 16, 16)] = jnp.where(
            w, jnp.float32(1.0), jnp.float32(0.0))

    pltpu.sync_copy(winpart_v, shared_win.at[s])
    plsc.subcore_barrier()

    @pl.when(c == 0)
    def _merge():
        jb = s * _JB
        for k in range(_NS):
            pltpu.sync_copy(shared_win.at[k, pl.ds(jb, _JB)],
                            tmp_v.at[pl.ds(k * _JB, _JB)])

        @plsc.parallel_loop(0, _JB // 16, unroll=4)
        def _or_loop(i):
            v = tmp_v[pl.ds(i * 16, 16)]
            for k in range(1, _NS):
                v = jnp.maximum(v, tmp_v[pl.ds(k * _JB + i * 16, 16)])
            acc_v[pl.ds(i * 16, 16)] = v

        pltpu.sync_copy(acc_v, win_hbm.at[pl.ds(jb, _JB)])

    for r in range(_ROWS):
        copies[r].wait()
        row = bufs[r % 3]

        @plsc.parallel_loop(0, _NCH, unroll=8)
        def _gather_loop(ci):
            k16 = idx_v[pl.ds(ci * 16, 16)]
            xgrow_v[pl.ds(ci * 16, 16)] = plsc.load_gather(row, [k16])
        pltpu.sync_copy(xgrow_v, xg_hbm.at[b0 + r])
        if r + 3 < _ROWS:
            copies[r + 3].start()


_route = pl.kernel(
    _route_body,
    out_type=(
        jax.ShapeDtypeStruct((L,), jnp.float32),
        jax.ShapeDtypeStruct((ND,), jnp.float32),
        jax.ShapeDtypeStruct((B, ND), jnp.float32),
    ),
    mesh=plsc.VectorSubcoreMesh(core_axis_name="c", subcore_axis_name="s"),
    compiler_params=pltpu.CompilerParams(needs_layout_passes=False),
    scratch_types=(
        pltpu.VMEM((ND,), jnp.int32),
        pltpu.VMEM((_SH,), jnp.int32),
        pltpu.VMEM((ND,), jnp.float32),
        pltpu.VMEM((_SH // 2,), jnp.float32),
        pltpu.VMEM((ND,), jnp.float32),
        pltpu.VMEM((_JB,), jnp.float32),
        pltpu.VMEM((L,), jnp.float32),
        pltpu.VMEM((L,), jnp.float32),
        pltpu.VMEM((L,), jnp.float32),
        pltpu.VMEM((ND,), jnp.float32),
        pltpu.SemaphoreType.DMA,
        pltpu.SemaphoreType.DMA,
        pltpu.SemaphoreType.DMA,
        pltpu.VMEM_SHARED((_NS, ND), jnp.float32),
    ),
)


_LB = 1024
_NMAIN = L // _LB
_NCORR = ND // _LB


def _tc_body(w_ref, u_ref, x_ref, xg_ref, keep_ref, win_ref, o_ref):
    i = pl.program_id(0)

    @pl.when(i < _NMAIN)
    def _main():
        w = w_ref[...]
        n2 = jnp.sum(w * w, axis=0, keepdims=True)
        sc = keep_ref[...] / jnp.maximum(jnp.sqrt(n2), 1e-8)
        xs = x_ref[...] * sc
        part = lax.dot_general(xs, w, (((1,), (1,)), ((), ())),
                               preferred_element_type=jnp.float32)

        @pl.when(i == 0)
        def _init():
            o_ref[...] = part

        @pl.when(i > 0)
        def _acc():
            o_ref[...] += part

    @pl.when(i >= _NMAIN)
    def _corr():
        u = u_ref[...]
        n2 = jnp.sum(u * u, axis=0, keepdims=True)
        sc = win_ref[...] / jnp.maximum(jnp.sqrt(n2), 1e-8)
        xs = xg_ref[...] * sc
        o_ref[...] += lax.dot_general(xs, u, (((1,), (1,)), ((), ())),
                                      preferred_element_type=jnp.float32)


def kernel(x, weight, dictionary_vector_indices, updated_weights):
    idx = dictionary_vector_indices.astype(jnp.int32)

    keep, win, xg = _route(idx, x)

    keep3 = keep.reshape(_NMAIN, 1, _LB)
    win3 = win.reshape(_NCORR, 1, _LB)
    out = pl.pallas_call(
        _tc_body,
        grid=(_NMAIN + _NCORR,),
        in_specs=[
            pl.BlockSpec((D, _LB), lambda i: (0, jnp.minimum(i, _NMAIN - 1))),
            pl.BlockSpec((D, _LB), lambda i: (0, jnp.maximum(i - _NMAIN, 0))),
            pl.BlockSpec((B, _LB), lambda i: (0, jnp.minimum(i, _NMAIN - 1))),
            pl.BlockSpec((B, _LB), lambda i: (0, jnp.maximum(i - _NMAIN, 0))),
            pl.BlockSpec((None, 1, _LB),
                         lambda i: (jnp.minimum(i, _NMAIN - 1), 0, 0)),
            pl.BlockSpec((None, 1, _LB),
                         lambda i: (jnp.maximum(i - _NMAIN, 0), 0, 0)),
        ],
        out_specs=pl.BlockSpec((B, D), lambda i: (0, 0)),
        out_shape=jax.ShapeDtypeStruct((B, D), jnp.float32),
    )(weight, updated_weights, x, xg, keep3, win3)
    return out


# speedup vs baseline: 1.0483x; 1.0483x over previous
import functools

import jax
import jax.numpy as jnp
from jax import lax
from jax.experimental import pallas as pl
from jax.experimental.pallas import tpu as pltpu
from jax.experimental.pallas import tpu_sc as plsc

B = 128
L = 32768
D = 2048
ND = 4096

_NC = 2
_NS = 16
_NW = _NC * _NS
_SH = L // _NS
_ROWS = B // _NW
_NCH = ND // 16
_JB = ND // _NS


def _route_body(idx_hbm, x_hbm, keep_hbm, win_hbm, xg_hbm,
                idx_v, shard_v, winpart_v, keep_v, tmp_v, acc_v,
                xrow0_v, xrow1_v, xrow2_v, xgrow_v,
                sem0, sem1, sem2, shared_win):
    c = lax.axis_index("c")
    s = lax.axis_index("s")
    base = s * _SH
    wid = s * _NC + c
    b0 = wid * _ROWS

    bufs = (xrow0_v, xrow1_v, xrow2_v)
    sems = (sem0, sem1, sem2)
    copies = [pltpu.make_async_copy(x_hbm.at[b0 + r], bufs[r % 3], sems[r % 3])
              for r in range(_ROWS)]
    for r in range(min(3, _ROWS)):
        copies[r].start()

    pltpu.sync_copy(idx_hbm, idx_v)

    @plsc.parallel_loop(0, _SH // 16, unroll=8)
    def _init_loop(i):
        shard_v[pl.ds(i * 16, 16)] = jnp.full((16,), -1, jnp.int32)

    def chunk_body(ci, carry):
        k16 = idx_v[pl.ds(ci * 16, 16)]
        j16 = lax.iota(jnp.int32, 16) + ci * 16
        m = (k16 >= base) & (k16 < base + _SH)
        loc = jnp.clip(k16 - base, 0, _SH - 1)
        cand = jnp.where(m, j16, -1)
        for _ in range(3):
            g = plsc.load_gather(shard_v, [loc], mask=m)
            need = m & (cand > g)
            plsc.store_scatter(shard_v, [loc], cand, mask=need)
        return carry
    lax.fori_loop(0, _NCH, chunk_body, 0, unroll=4)

    koff = c * (_SH // 2)

    @plsc.parallel_loop(0, _SH // 32, unroll=8)
    def _keep_loop(i):
        a16 = shard_v[pl.ds(koff + i * 16, 16)]
        keep_v[pl.ds(i * 16, 16)] = jnp.where(
            a16 == -1, jnp.float32(1.0), jnp.float32(0.0))

    pltpu.sync_copy(keep_v, keep_hbm.at[pl.ds(base + koff, _SH // 2)])

    @plsc.parallel_loop(0, _NCH, unroll=8)
    def _winpart_loop(ci):
        k16 = idx_v[pl.ds(ci * 16, 16)]
        j16 = lax.iota(jnp.int32, 16) + ci * 16
        m = (k16 >= base) & (k16 < base + _SH)
        loc = jnp.clip(k16 - base, 0, _SH - 1)
        g = plsc.load_gather(shard_v, [loc], mask=m)
        w = m & (g == j16)
        winpart_v[pl.ds(ci * 16, 16)] = jnp.where(
            w, jnp.float32(1.0), jnp.float32(0.0))

    pltpu.sync_copy(winpart_v, shared_win.at[s])
    plsc.subcore_barrier()

    @pl.when(c == 0)
    def _merge():
        jb = s * _JB
        for k in range(_NS):
            pltpu.sync_copy(shared_win.at[k, pl.ds(jb, _JB)],
                            tmp_v.at[pl.ds(k * _JB, _JB)])

        @plsc.parallel_loop(0, _JB // 16, unroll=4)
        def _or_loop(i):
            v = tmp_v[pl.ds(i * 16, 16)]
            for k in range(1, _NS):
                v = jnp.maximum(v, tmp_v[pl.ds(k * _JB + i * 16, 16)])
            acc_v[pl.ds(i * 16, 16)] = v

        pltpu.sync_copy(acc_v, win_hbm.at[pl.ds(jb, _JB)])

    for r in range(_ROWS):
        copies[r].wait()
        row = bufs[r % 3]

        @plsc.parallel_loop(0, _NCH, unroll=8)
        def _gather_loop(ci):
            k16 = idx_v[pl.ds(ci * 16, 16)]
            xgrow_v[pl.ds(ci * 16, 16)] = plsc.load_gather(row, [k16])
        pltpu.sync_copy(xgrow_v, xg_hbm.at[b0 + r])
        if r + 3 < _ROWS:
            copies[r + 3].start()


_route = pl.kernel(
    _route_body,
    out_type=(
        jax.ShapeDtypeStruct((L,), jnp.float32),
        jax.ShapeDtypeStruct((ND,), jnp.float32),
        jax.ShapeDtypeStruct((B, ND), jnp.float32),
    ),
    mesh=plsc.VectorSubcoreMesh(core_axis_name="c", subcore_axis_name="s"),
    compiler_params=pltpu.CompilerParams(needs_layout_passes=False),
    scratch_types=(
        pltpu.VMEM((ND,), jnp.int32),
        pltpu.VMEM((_SH,), jnp.int32),
        pltpu.VMEM((ND,), jnp.float32),
        pltpu.VMEM((_SH // 2,), jnp.float32),
        pltpu.VMEM((ND,), jnp.float32),
        pltpu.VMEM((_JB,), jnp.float32),
        pltpu.VMEM((L,), jnp.float32),
        pltpu.VMEM((L,), jnp.float32),
        pltpu.VMEM((L,), jnp.float32),
        pltpu.VMEM((ND,), jnp.float32),
        pltpu.SemaphoreType.DMA,
        pltpu.SemaphoreType.DMA,
        pltpu.SemaphoreType.DMA,
        pltpu.VMEM_SHARED((_NS, ND), jnp.float32),
    ),
)


def _main_body(w_ref, x_ref, keep_ref, o_ref):
    i = pl.program_id(0)
    w = w_ref[...]
    n2 = jnp.sum(w * w, axis=0, keepdims=True)
    sc = keep_ref[...] / jnp.maximum(jnp.sqrt(n2), 1e-8)
    xs = x_ref[...] * sc
    part = lax.dot_general(xs, w, (((1,), (1,)), ((), ())),
                           preferred_element_type=jnp.float32)

    @pl.when(i == 0)
    def _init():
        o_ref[...] = part

    @pl.when(i > 0)
    def _acc():
        o_ref[...] += part


def _corr_body(u_ref, xg_ref, win_ref, acc_ref, o_ref):
    i = pl.program_id(0)
    u = u_ref[...]
    n2 = jnp.sum(u * u, axis=0, keepdims=True)
    sc = win_ref[...] / jnp.maximum(jnp.sqrt(n2), 1e-8)
    xs = xg_ref[...] * sc
    part = lax.dot_general(xs, u, (((1,), (1,)), ((), ())),
                           preferred_element_type=jnp.float32)

    @pl.when(i == 0)
    def _init():
        o_ref[...] = acc_ref[...] + part

    @pl.when(i > 0)
    def _acc():
        o_ref[...] += part


def kernel(x, weight, dictionary_vector_indices, updated_weights):
    idx = dictionary_vector_indices.astype(jnp.int32)

    keep, win, xg = _route(idx, x)

    LB = 2048
    nL = L // LB
    keep3 = keep.reshape(nL, 1, LB)
    acc1 = pl.pallas_call(
        _main_body,
        grid=(nL,),
        in_specs=[
            pl.BlockSpec((D, LB), lambda i: (0, i)),
            pl.BlockSpec((B, LB), lambda i: (0, i)),
            pl.BlockSpec((None, 1, LB), lambda i: (i, 0, 0)),
        ],
        out_specs=pl.BlockSpec((B, D), lambda i: (0, 0)),
        out_shape=jax.ShapeDtypeStruct((B, D), jnp.float32),
    )(weight, x, keep3)

    NB = 1024
    nN = ND // NB
    win3 = win.reshape(nN, 1, NB)
    out = pl.pallas_call(
        _corr_body,
        grid=(nN,),
        in_specs=[
            pl.BlockSpec((D, NB), lambda i: (0, i)),
            pl.BlockSpec((B, NB), lambda i: (0, i)),
            pl.BlockSpec((None, 1, NB), lambda i: (i, 0, 0)),
            pl.BlockSpec((B, D), lambda i: (0, 0)),
        ],
        out_specs=pl.BlockSpec((B, D), lambda i: (0, 0)),
        out_shape=jax.ShapeDtypeStruct((B, D), jnp.float32),
    )(updated_weights, xg, win3, acc1)
    return out


# speedup vs baseline: 1.0543x; 1.0057x over previous
import functools

import jax
import jax.numpy as jnp
from jax import lax
from jax.experimental import pallas as pl
from jax.experimental.pallas import tpu as pltpu
from jax.experimental.pallas import tpu_sc as plsc

B = 128
L = 32768
D = 2048
ND = 4096

_NC = 2
_NS = 16
_NW = _NC * _NS
_SH = L // _NS
_ROWS = B // _NW
_NCH = ND // 16
_JB = ND // _NS


def _route_body(idx_hbm, x_hbm, keep_hbm, win_hbm, xg_hbm,
                idx_v, shard_v, winpart_v, keep_v, tmp_v, acc_v,
                xrow0_v, xrow1_v, xrow2_v, xgrow_v,
                sem0, sem1, sem2, shared_win):
    c = lax.axis_index("c")
    s = lax.axis_index("s")
    base = s * _SH
    wid = s * _NC + c
    b0 = wid * _ROWS

    bufs = (xrow0_v, xrow1_v, xrow2_v)
    sems = (sem0, sem1, sem2)
    copies = [pltpu.make_async_copy(x_hbm.at[b0 + r], bufs[r % 3], sems[r % 3])
              for r in range(_ROWS)]
    for r in range(min(3, _ROWS)):
        copies[r].start()

    pltpu.sync_copy(idx_hbm, idx_v)

    @plsc.parallel_loop(0, _SH // 16, unroll=8)
    def _init_loop(i):
        shard_v[pl.ds(i * 16, 16)] = jnp.full((16,), -1, jnp.int32)

    def chunk_body(ci, carry):
        k16 = idx_v[pl.ds(ci * 16, 16)]
        j16 = lax.iota(jnp.int32, 16) + ci * 16
        m = (k16 >= base) & (k16 < base + _SH)
        loc = jnp.clip(k16 - base, 0, _SH - 1)
        cand = jnp.where(m, j16, -1)
        for _ in range(2):
            g = plsc.load_gather(shard_v, [loc], mask=m)
            need = m & (cand > g)
            plsc.store_scatter(shard_v, [loc], cand, mask=need)
        return carry
    lax.fori_loop(0, _NCH, chunk_body, 0, unroll=8)

    koff = c * (_SH // 2)

    @plsc.parallel_loop(0, _SH // 32, unroll=8)
    def _keep_loop(i):
        a16 = shard_v[pl.ds(koff + i * 16, 16)]
        keep_v[pl.ds(i * 16, 16)] = jnp.where(
            a16 == -1, jnp.float32(1.0), jnp.float32(0.0))

    pltpu.sync_copy(keep_v, keep_hbm.at[pl.ds(base + koff, _SH // 2)])

    @plsc.parallel_loop(0, _NCH, unroll=8)
    def _winpart_loop(ci):
        k16 = idx_v[pl.ds(ci * 16, 16)]
        j16 = lax.iota(jnp.int32, 16) + ci * 16
        m = (k16 >= base) & (k16 < base + _SH)
        loc = jnp.clip(k16 - base, 0, _SH - 1)
        g = plsc.load_gather(shard_v, [loc], mask=m)
        w = m & (g == j16)
        winpart_v[pl.ds(ci * 16, 16)] = jnp.where(
            w, jnp.float32(1.0), jnp.float32(0.0))

    pltpu.sync_copy(winpart_v, shared_win.at[s])
    plsc.subcore_barrier()

    @pl.when(c == 0)
    def _merge():
        jb = s * _JB
        for k in range(_NS):
            pltpu.sync_copy(shared_win.at[k, pl.ds(jb, _JB)],
                            tmp_v.at[pl.ds(k * _JB, _JB)])

        @plsc.parallel_loop(0, _JB // 16, unroll=4)
        def _or_loop(i):
            v = tmp_v[pl.ds(i * 16, 16)]
            for k in range(1, _NS):
                v = jnp.maximum(v, tmp_v[pl.ds(k * _JB + i * 16, 16)])
            acc_v[pl.ds(i * 16, 16)] = v

        pltpu.sync_copy(acc_v, win_hbm.at[pl.ds(jb, _JB)])

    for r in range(_ROWS):
        copies[r].wait()
        row = bufs[r % 3]

        @plsc.parallel_loop(0, _NCH, unroll=8)
        def _gather_loop(ci):
            k16 = idx_v[pl.ds(ci * 16, 16)]
            xgrow_v[pl.ds(ci * 16, 16)] = plsc.load_gather(row, [k16])
        pltpu.sync_copy(xgrow_v, xg_hbm.at[b0 + r])
        if r + 3 < _ROWS:
            copies[r + 3].start()


_route = pl.kernel(
    _route_body,
    out_type=(
        jax.ShapeDtypeStruct((L,), jnp.float32),
        jax.ShapeDtypeStruct((ND,), jnp.float32),
        jax.ShapeDtypeStruct((B, ND), jnp.float32),
    ),
    mesh=plsc.VectorSubcoreMesh(core_axis_name="c", subcore_axis_name="s"),
    compiler_params=pltpu.CompilerParams(needs_layout_passes=False),
    scratch_types=(
        pltpu.VMEM((ND,), jnp.int32),
        pltpu.VMEM((_SH,), jnp.int32),
        pltpu.VMEM((ND,), jnp.float32),
        pltpu.VMEM((_SH // 2,), jnp.float32),
        pltpu.VMEM((ND,), jnp.float32),
        pltpu.VMEM((_JB,), jnp.float32),
        pltpu.VMEM((L,), jnp.float32),
        pltpu.VMEM((L,), jnp.float32),
        pltpu.VMEM((L,), jnp.float32),
        pltpu.VMEM((ND,), jnp.float32),
        pltpu.SemaphoreType.DMA,
        pltpu.SemaphoreType.DMA,
        pltpu.SemaphoreType.DMA,
        pltpu.VMEM_SHARED((_NS, ND), jnp.float32),
    ),
)


def _main_body(w_ref, x_ref, keep_ref, o_ref):
    i = pl.program_id(0)
    w = w_ref[...]
    n2 = jnp.sum(w * w, axis=0, keepdims=True)
    sc = keep_ref[...] / jnp.maximum(jnp.sqrt(n2), 1e-8)
    xs = x_ref[...] * sc
    part = lax.dot_general(xs, w, (((1,), (1,)), ((), ())),
                           preferred_element_type=jnp.float32)

    @pl.when(i == 0)
    def _init():
        o_ref[...] = part

    @pl.when(i > 0)
    def _acc():
        o_ref[...] += part


def _corr_body(u_ref, xg_ref, win_ref, acc_ref, o_ref):
    i = pl.program_id(0)
    u = u_ref[...]
    n2 = jnp.sum(u * u, axis=0, keepdims=True)
    sc = win_ref[...] / jnp.maximum(jnp.sqrt(n2), 1e-8)
    xs = xg_ref[...] * sc
    part = lax.dot_general(xs, u, (((1,), (1,)), ((), ())),
                           preferred_element_type=jnp.float32)

    @pl.when(i == 0)
    def _init():
        o_ref[...] = acc_ref[...] + part

    @pl.when(i > 0)
    def _acc():
        o_ref[...] += part


def kernel(x, weight, dictionary_vector_indices, updated_weights):
    idx = dictionary_vector_indices.astype(jnp.int32)

    keep, win, xg = _route(idx, x)

    LB = 2048
    nL = L // LB
    keep3 = keep.reshape(nL, 1, LB)
    acc1 = pl.pallas_call(
        _main_body,
        grid=(nL,),
        in_specs=[
            pl.BlockSpec((D, LB), lambda i: (0, i)),
            pl.BlockSpec((B, LB), lambda i: (0, i)),
            pl.BlockSpec((None, 1, LB), lambda i: (i, 0, 0)),
        ],
        out_specs=pl.BlockSpec((B, D), lambda i: (0, 0)),
        out_shape=jax.ShapeDtypeStruct((B, D), jnp.float32),
    )(weight, x, keep3)

    NB = 2048
    nN = ND // NB
    win3 = win.reshape(nN, 1, NB)
    out = pl.pallas_call(
        _corr_body,
        grid=(nN,),
        in_specs=[
            pl.BlockSpec((D, NB), lambda i: (0, i)),
            pl.BlockSpec((B, NB), lambda i: (0, i)),
            pl.BlockSpec((None, 1, NB), lambda i: (i, 0, 0)),
            pl.BlockSpec((B, D), lambda i: (0, 0)),
        ],
        out_specs=pl.BlockSpec((B, D), lambda i: (0, 0)),
        out_shape=jax.ShapeDtypeStruct((B, D), jnp.float32),
    )(updated_weights, xg, win3, acc1)
    return out


# speedup vs baseline: 1.0648x; 1.0099x over previous
import functools

import jax
import jax.numpy as jnp
from jax import lax
from jax.experimental import pallas as pl
from jax.experimental.pallas import tpu as pltpu
from jax.experimental.pallas import tpu_sc as plsc

B = 128
L = 32768
D = 2048
ND = 4096

_NC = 2
_NS = 16
_NW = _NC * _NS
_SH = L // _NS
_ROWS = B // _NW
_NCH = ND // 16
_JB = ND // _NS
_JH = _JB // 2


def _route_body(idx_hbm, x_hbm, keep_hbm, win_hbm, xg_hbm,
                idx_v, shard_v, winpart_v, keep_v, tmp_v, acc_v,
                xrow0_v, xrow1_v, xrow2_v, xgrow_v,
                sem0, sem1, sem2, msem, shared_win):
    c = lax.axis_index("c")
    s = lax.axis_index("s")
    base = s * _SH
    wid = s * _NC + c
    b0 = wid * _ROWS

    bufs = (xrow0_v, xrow1_v, xrow2_v)
    sems = (sem0, sem1, sem2)
    copies = [pltpu.make_async_copy(x_hbm.at[b0 + r], bufs[r % 3], sems[r % 3])
              for r in range(_ROWS)]
    for r in range(min(3, _ROWS)):
        copies[r].start()

    pltpu.sync_copy(idx_hbm, idx_v)

    @plsc.parallel_loop(0, _SH // 16, unroll=8)
    def _init_loop(i):
        shard_v[pl.ds(i * 16, 16)] = jnp.full((16,), -1, jnp.int32)

    def chunk_body(ci, carry):
        k16 = idx_v[pl.ds(ci * 16, 16)]
        j16 = lax.iota(jnp.int32, 16) + ci * 16
        m = (k16 >= base) & (k16 < base + _SH)
        loc = jnp.clip(k16 - base, 0, _SH - 1)
        cand = jnp.where(m, j16, -1)
        for _ in range(2):
            g = plsc.load_gather(shard_v, [loc], mask=m)
            need = m & (cand > g)
            plsc.store_scatter(shard_v, [loc], cand, mask=need)
        return carry
    lax.fori_loop(0, _NCH, chunk_body, 0, unroll=8)

    koff = c * (_SH // 2)

    @plsc.parallel_loop(0, _SH // 32, unroll=8)
    def _keep_loop(i):
        a16 = shard_v[pl.ds(koff + i * 16, 16)]
        keep_v[pl.ds(i * 16, 16)] = jnp.where(
            a16 == -1, jnp.float32(1.0), jnp.float32(0.0))

    pltpu.sync_copy(keep_v, keep_hbm.at[pl.ds(base + koff, _SH // 2)])

    @plsc.parallel_loop(0, _NCH, unroll=8)
    def _winpart_loop(ci):
        k16 = idx_v[pl.ds(ci * 16, 16)]
        j16 = lax.iota(jnp.int32, 16) + ci * 16
        m = (k16 >= base) & (k16 < base + _SH)
        loc = jnp.clip(k16 - base, 0, _SH - 1)
        g = plsc.load_gather(shard_v, [loc], mask=m)
        w = m & (g == j16)
        winpart_v[pl.ds(ci * 16, 16)] = jnp.where(
            w, jnp.float32(1.0), jnp.float32(0.0))

    pltpu.sync_copy(winpart_v, shared_win.at[s])
    plsc.subcore_barrier()

    jb = s * _JB + c * _JH
    mcopies = [
        pltpu.make_async_copy(shared_win.at[k, pl.ds(jb, _JH)],
                              tmp_v.at[pl.ds(k * _JH, _JH)], msem)
        for k in range(_NS)
    ]
    for k in range(_NS):
        mcopies[k].start()
    for k in range(_NS):
        mcopies[k].wait()

    @plsc.parallel_loop(0, _JH // 16, unroll=4)
    def _or_loop(i):
        v = tmp_v[pl.ds(i * 16, 16)]
        for k in range(1, _NS):
            v = jnp.maximum(v, tmp_v[pl.ds(k * _JH + i * 16, 16)])
        acc_v[pl.ds(i * 16, 16)] = v

    pltpu.sync_copy(acc_v, win_hbm.at[pl.ds(jb, _JH)])

    for r in range(_ROWS):
        copies[r].wait()
        row = bufs[r % 3]

        @plsc.parallel_loop(0, _NCH, unroll=8)
        def _gather_loop(ci):
            k16 = idx_v[pl.ds(ci * 16, 16)]
            xgrow_v[pl.ds(ci * 16, 16)] = plsc.load_gather(row, [k16])
        pltpu.sync_copy(xgrow_v, xg_hbm.at[b0 + r])
        if r + 3 < _ROWS:
            copies[r + 3].start()


_route = pl.kernel(
    _route_body,
    out_type=(
        jax.ShapeDtypeStruct((L,), jnp.float32),
        jax.ShapeDtypeStruct((ND,), jnp.float32),
        jax.ShapeDtypeStruct((B, ND), jnp.float32),
    ),
    mesh=plsc.VectorSubcoreMesh(core_axis_name="c", subcore_axis_name="s"),
    compiler_params=pltpu.CompilerParams(needs_layout_passes=False),
    scratch_types=(
        pltpu.VMEM((ND,), jnp.int32),
        pltpu.VMEM((_SH,), jnp.int32),
        pltpu.VMEM((ND,), jnp.float32),
        pltpu.VMEM((_SH // 2,), jnp.float32),
        pltpu.VMEM((_NS * _JH,), jnp.float32),
        pltpu.VMEM((_JH,), jnp.float32),
        pltpu.VMEM((L,), jnp.float32),
        pltpu.VMEM((L,), jnp.float32),
        pltpu.VMEM((L,), jnp.float32),
        pltpu.VMEM((ND,), jnp.float32),
        pltpu.SemaphoreType.DMA,
        pltpu.SemaphoreType.DMA,
        pltpu.SemaphoreType.DMA,
        pltpu.SemaphoreType.DMA,
        pltpu.VMEM_SHARED((_NS, ND), jnp.float32),
    ),
)


def _main_body(w_ref, x_ref, keep_ref, o_ref):
    i = pl.program_id(0)
    w = w_ref[...]
    n2 = jnp.sum(w * w, axis=0, keepdims=True)
    sc = keep_ref[...] / jnp.maximum(jnp.sqrt(n2), 1e-8)
    xs = x_ref[...] * sc
    part = lax.dot_general(xs, w, (((1,), (1,)), ((), ())),
                           preferred_element_type=jnp.float32)

    @pl.when(i == 0)
    def _init():
        o_ref[...] = part

    @pl.when(i > 0)
    def _acc():
        o_ref[...] += part


def _corr_body(u_ref, xg_ref, win_ref, acc_ref, o_ref):
    i = pl.program_id(0)
    u = u_ref[...]
    n2 = jnp.sum(u * u, axis=0, keepdims=True)
    sc = win_ref[...] / jnp.maximum(jnp.sqrt(n2), 1e-8)
    xs = xg_ref[...] * sc
    part = lax.dot_general(xs, u, (((1,), (1,)), ((), ())),
                           preferred_element_type=jnp.float32)

    @pl.when(i == 0)
    def _init():
        o_ref[...] = acc_ref[...] + part

    @pl.when(i > 0)
    def _acc():
        o_ref[...] += part


def kernel(x, weight, dictionary_vector_indices, updated_weights):
    idx = dictionary_vector_indices.astype(jnp.int32)

    keep, win, xg = _route(idx, x)

    LB = 2048
    nL = L // LB
    keep3 = keep.reshape(nL, 1, LB)
    acc1 = pl.pallas_call(
        _main_body,
        grid=(nL,),
        in_specs=[
            pl.BlockSpec((D, LB), lambda i: (0, i)),
            pl.BlockSpec((B, LB), lambda i: (0, i)),
            pl.BlockSpec((None, 1, LB), lambda i: (i, 0, 0)),
        ],
        out_specs=pl.BlockSpec((B, D), lambda i: (0, 0)),
        out_shape=jax.ShapeDtypeStruct((B, D), jnp.float32),
    )(weight, x, keep3)

    NB = 2048
    nN = ND // NB
    win3 = win.reshape(nN, 1, NB)
    out = pl.pallas_call(
        _corr_body,
        grid=(nN,),
        in_specs=[
            pl.BlockSpec((D, NB), lambda i: (0, i)),
            pl.BlockSpec((B, NB), lambda i: (0, i)),
            pl.BlockSpec((None, 1, NB), lambda i: (i, 0, 0)),
            pl.BlockSpec((B, D), lambda i: (0, 0)),
        ],
        out_specs=pl.BlockSpec((B, D), lambda i: (0, 0)),
        out_shape=jax.ShapeDtypeStruct((B, D), jnp.float32),
    )(updated_weights, xg, win3, acc1)
    return out
